# baseline (speedup 1.0000x reference)
import functools

import jax
import jax.numpy as jnp
import numpy as np
from jax import lax
from jax.experimental import pallas as pl
from jax.experimental.pallas import tpu as pltpu
from jax.experimental.pallas import tpu_sc as plsc

_NUM_CLUSTERS = 64
_NC = 2
_NS = 16
_NW = _NC * _NS


@functools.lru_cache(maxsize=None)
def _flat_indices(n_rows: int, seq_len: int, n_clusters: int):
    ind = (np.arange(n_clusters, dtype=np.int64) * (seq_len - 1)) // (n_clusters - 1)
    r = np.arange(n_rows, dtype=np.int64)[:, None]
    c = ind[None, :]
    n_ctiles = seq_len // 128
    flat = ((r // 8) * n_ctiles + c // 128) * 1024 + (r % 8) * 128 + (c % 128)
    per_w = n_rows * n_clusters // _NW
    return np.ascontiguousarray(flat.reshape(_NW, per_w).astype(np.int32))


def _tiled_flat_view(A):
    n_rows, seq_len = A.shape
    x = A.reshape(n_rows // 8, 8, seq_len // 128, 128)
    return x.transpose(0, 2, 1, 3).reshape(-1)


def _sc_gather(Kf, Vf, idx, *, per_w: int):
    mesh = plsc.VectorSubcoreMesh(core_axis_name="c", subcore_axis_name="s")

    @functools.partial(
        pl.kernel,
        out_type=(
            jax.ShapeDtypeStruct((_NW * per_w // 64, 64), jnp.float32),
            jax.ShapeDtypeStruct((_NW * per_w // 64, 64), jnp.float32),
        ),
        mesh=mesh,
        scratch_types=[
            pltpu.VMEM((per_w,), jnp.int32),
            pltpu.VMEM((per_w,), jnp.float32),
            pltpu.VMEM((per_w,), jnp.float32),
            pltpu.VMEM((per_w // 64, 64), jnp.float32),
            pltpu.VMEM((per_w // 64, 64), jnp.float32),
            pltpu.SemaphoreType.DMA,
            pltpu.SemaphoreType.DMA,
        ],
    )
    def k(k_hbm, v_hbm, idx_hbm, outk_hbm, outv_hbm, idx_v, gk_v, gv_v,
          gk2_v, gv2_v, sem_k, sem_v):
        wid = lax.axis_index("s") * _NC + lax.axis_index("c")
        pltpu.sync_copy(idx_hbm.at[wid], idx_v)
        cp_k = pltpu.make_async_copy(k_hbm.at[idx_v], gk_v, sem_k)
        cp_v = pltpu.make_async_copy(v_hbm.at[idx_v], gv_v, sem_v)
        cp_k.start()
        cp_v.start()
        cp_k.wait()
        cp_v.wait()
        rows_per_w = per_w // 64

        def relayout(i, _):
            for j in range(4):
                sl = pl.ds(j * 16, 16)
                gk2_v[i, sl] = gk_v[pl.ds(i * 64 + j * 16, 16)]
                gv2_v[i, sl] = gv_v[pl.ds(i * 64 + j * 16, 16)]
            return 0

        lax.fori_loop(0, rows_per_w, relayout, 0)
        base = wid * rows_per_w
        pltpu.sync_copy(gk2_v, outk_hbm.at[pl.ds(base, rows_per_w)])
        pltpu.sync_copy(gv2_v, outv_hbm.at[pl.ds(base, rows_per_w)])

    return k(Kf, Vf, idx)


def kernel(K, V):
    n_rows, seq_len = K.shape
    n_clusters = min(_NUM_CLUSTERS, seq_len)
    idx_np = _flat_indices(n_rows, seq_len, n_clusters)
    per_w = n_rows * n_clusters // _NW
    outk, outv = _sc_gather(
        _tiled_flat_view(K), _tiled_flat_view(V), jnp.asarray(idx_np),
        per_w=per_w)
    return (outk, outv)


# speedup vs baseline: 1.0861x; 1.0861x over previous
import functools

import jax
import jax.numpy as jnp
import numpy as np
from jax import lax
from jax.experimental import pallas as pl
from jax.experimental.pallas import tpu as pltpu
from jax.experimental.pallas import tpu_sc as plsc

_NUM_CLUSTERS = 64
_NC = 2
_NS = 16
_NW = _NC * _NS


@functools.lru_cache(maxsize=None)
def _flat_indices(n_rows: int, seq_len: int, n_clusters: int):
    ind = (np.arange(n_clusters, dtype=np.int64) * (seq_len - 1)) // (n_clusters - 1)
    r = np.arange(n_rows, dtype=np.int64)[:, None]
    c = ind[None, :]
    n_ctiles = seq_len // 128
    flat = ((r // 8) * n_ctiles + c // 128) * 1024 + (r % 8) * 128 + (c % 128)
    per_w = n_rows * n_clusters // _NW
    return np.ascontiguousarray(flat.reshape(_NW, per_w).astype(np.int32))


def _tiled_flat_view(A):
    n_rows, seq_len = A.shape
    x = A.reshape(n_rows // 8, 8, seq_len // 128, 128)
    return x.transpose(0, 2, 1, 3).reshape(-1)


def _sc_gather(Kf, Vf, idx, *, per_w: int):
    mesh = plsc.VectorSubcoreMesh(core_axis_name="c", subcore_axis_name="s")

    @functools.partial(
        pl.kernel,
        out_type=(
            jax.ShapeDtypeStruct((_NW * per_w,), jnp.float32),
            jax.ShapeDtypeStruct((_NW * per_w,), jnp.float32),
        ),
        mesh=mesh,
        scratch_types=[
            pltpu.VMEM((per_w,), jnp.int32),
            pltpu.VMEM((per_w,), jnp.float32),
            pltpu.VMEM((per_w,), jnp.float32),
            pltpu.SemaphoreType.DMA,
            pltpu.SemaphoreType.DMA,
        ],
    )
    def k(k_hbm, v_hbm, idx_hbm, outk_hbm, outv_hbm, idx_v, gk_v, gv_v,
          sem_k, sem_v):
        wid = lax.axis_index("s") * _NC + lax.axis_index("c")
        pltpu.sync_copy(idx_hbm.at[wid], idx_v)
        cp_k = pltpu.make_async_copy(k_hbm.at[idx_v], gk_v, sem_k)
        cp_v = pltpu.make_async_copy(v_hbm.at[idx_v], gv_v, sem_v)
        cp_k.start()
        cp_v.start()
        cp_k.wait()
        cp_v.wait()
        base = wid * per_w
        pltpu.sync_copy(gk_v, outk_hbm.at[pl.ds(base, per_w)])
        pltpu.sync_copy(gv_v, outv_hbm.at[pl.ds(base, per_w)])

    return k(Kf, Vf, idx)


def kernel(K, V):
    n_rows, seq_len = K.shape
    n_clusters = min(_NUM_CLUSTERS, seq_len)
    idx_np = _flat_indices(n_rows, seq_len, n_clusters)
    per_w = n_rows * n_clusters // _NW
    outk, outv = _sc_gather(
        _tiled_flat_view(K), _tiled_flat_view(V), jnp.asarray(idx_np),
        per_w=per_w)
    return (outk.reshape(n_rows, n_clusters), outv.reshape(n_rows, n_clusters))
